# Initial kernel scaffold; baseline (speedup 1.0000x reference)
import jax
import jax.numpy as jnp
from jax.experimental import pallas as pl


def kernel(x, edge_index, W1, a1, W2, a2):
    raise NotImplementedError("write your pallas kernel here")


# speedup vs baseline: 37.6331x; 37.6331x over previous
import functools

import jax
import jax.numpy as jnp
from jax import lax
from jax.experimental import pallas as pl
from jax.experimental.pallas import tpu as pltpu
from jax.experimental.pallas import tpu_sc as plsc

N = 10000
E = 320000
NFEAT = 128
NHID = 16
NCLASS = 16
NHEADS = 8
ALPHA = 0.2

NC, NS, L = 2, 16, 16
NW = NC * NS
EPW = E // NW
C = 80
NCHUNK = EPW // C
RPT = N // NS

_f32 = jnp.float32
_i32 = jnp.int32


def _leaky_exp(e):
    return jnp.exp(jnp.where(e >= 0, e, ALPHA * e))


def _tc1_body(x_ref, w_ref, as_ref, ad_ref, h_ref, s_ref, d_ref):
    h = jnp.dot(x_ref[...], w_ref[...], preferred_element_type=_f32)
    h_ref[...] = h
    s = jnp.dot(h, as_ref[...], preferred_element_type=_f32)
    d = jnp.dot(h, ad_ref[...], preferred_element_type=_f32)
    s_ref[...] = jnp.concatenate([s, s], axis=1)
    d_ref[...] = jnp.concatenate([d, d], axis=1)


def _tc1(x, w1cat, a_src, a_dst):
    B = 2000
    grid = (N // B,)
    return pl.pallas_call(
        _tc1_body,
        grid=grid,
        in_specs=[
            pl.BlockSpec((B, NFEAT), lambda i: (i, 0)),
            pl.BlockSpec((NFEAT, NFEAT), lambda i: (0, 0)),
            pl.BlockSpec((NFEAT, NHEADS), lambda i: (0, 0)),
            pl.BlockSpec((NFEAT, NHEADS), lambda i: (0, 0)),
        ],
        out_specs=[
            pl.BlockSpec((B, NFEAT), lambda i: (i, 0)),
            pl.BlockSpec((B, 2 * NHEADS), lambda i: (i, 0)),
            pl.BlockSpec((B, 2 * NHEADS), lambda i: (i, 0)),
        ],
        out_shape=[
            jax.ShapeDtypeStruct((N, NFEAT), _f32),
            jax.ShapeDtypeStruct((N, 2 * NHEADS), _f32),
            jax.ShapeDtypeStruct((N, 2 * NHEADS), _f32),
        ],
    )(x, w1cat, a_src, a_dst)


def _tc2_body(acc_ref, den_ref, w2_ref, a2s_ref, a2d_ref, r_ref,
              h2_ref, s2_ref, d2_ref):
    o = acc_ref[0] + acc_ref[1]
    dn = den_ref[0] + den_ref[1]
    r = 1.0 / (dn[:, :NHEADS] + 1e-9)
    db = jnp.dot(r, r_ref[...], preferred_element_type=_f32)
    on = o * db
    el = jnp.where(on > 0, on, jnp.exp(jnp.minimum(on, 0.0)) - 1.0)
    h2 = jnp.dot(el, w2_ref[...], preferred_element_type=_f32)
    h2_ref[...] = h2
    s2_ref[...] = jnp.dot(h2, a2s_ref[...], preferred_element_type=_f32)
    d2_ref[...] = jnp.dot(h2, a2d_ref[...], preferred_element_type=_f32)


def _tc2(acc1, den1, w2, a2s, a2d, rmat):
    B = 2000
    grid = (N // B,)
    return pl.pallas_call(
        _tc2_body,
        grid=grid,
        in_specs=[
            pl.BlockSpec((NC, B, NFEAT), lambda i: (0, i, 0)),
            pl.BlockSpec((NC, B, 2 * NHEADS), lambda i: (0, i, 0)),
            pl.BlockSpec((NFEAT, NCLASS), lambda i: (0, 0)),
            pl.BlockSpec((NCLASS, NCLASS), lambda i: (0, 0)),
            pl.BlockSpec((NCLASS, NCLASS), lambda i: (0, 0)),
            pl.BlockSpec((NHEADS, NFEAT), lambda i: (0, 0)),
        ],
        out_specs=[
            pl.BlockSpec((B, NCLASS), lambda i: (i, 0)),
            pl.BlockSpec((B, NCLASS), lambda i: (i, 0)),
            pl.BlockSpec((B, NCLASS), lambda i: (i, 0)),
        ],
        out_shape=[
            jax.ShapeDtypeStruct((N, NCLASS), _f32),
            jax.ShapeDtypeStruct((N, NCLASS), _f32),
            jax.ShapeDtypeStruct((N, NCLASS), _f32),
        ],
    )(acc1, den1, w2, a2s, a2d, rmat)


def _tc3_body(acc_ref, den_ref, out_ref):
    o = acc_ref[0] + acc_ref[1]
    dn = den_ref[0][:, 0:1] + den_ref[1][:, 0:1] + 1e-9
    out_ref[...] = o / dn


def _tc3(acc2, den2):
    B = 2000
    grid = (N // B,)
    return pl.pallas_call(
        _tc3_body,
        grid=grid,
        in_specs=[
            pl.BlockSpec((NC, B, NCLASS), lambda i: (0, i, 0)),
            pl.BlockSpec((NC, B, NCLASS), lambda i: (0, i, 0)),
        ],
        out_specs=pl.BlockSpec((B, NCLASS), lambda i: (i, 0)),
        out_shape=jax.ShapeDtypeStruct((N, NCLASS), _f32),
    )(acc2, den2)


def _sc1_body(src_hbm, dst_hbm, h_hbm, as_hbm, ad_hbm, z128_hbm, z16_hbm,
              out_hbm, den_hbm,
              src_v, dst_v, gs_v, gd_v, w_v, rows_v,
              sh_out, sh_den, sem0, sem1, sem2):
    cid = lax.axis_index("c")
    sid = lax.axis_index("s")
    wid = sid * NC + cid
    r0 = sid * 1000

    @pl.when(sid < 10)
    def _():
        pltpu.sync_copy(z128_hbm.at[pl.ds(r0, 1000)],
                        sh_out.at[pl.ds(r0, 1000)])
        pltpu.sync_copy(z16_hbm.at[pl.ds(r0, 1000)],
                        sh_den.at[pl.ds(r0, 1000)])

    plsc.subcore_barrier()

    def chunk(k, carry):
        base = wid * EPW + k * C
        pltpu.sync_copy(src_hbm.at[pl.ds(base, C)], src_v)
        pltpu.sync_copy(dst_hbm.at[pl.ds(base, C)], dst_v)
        cg0 = pltpu.async_copy(h_hbm.at[src_v], rows_v, sem0)
        cg1 = pltpu.async_copy(as_hbm.at[src_v], gs_v, sem1)
        cg2 = pltpu.async_copy(ad_hbm.at[dst_v], gd_v, sem2)
        cg1.wait()
        cg2.wait()

        def edge_w(i, carry2):
            e = gs_v.at[i][...] + gd_v.at[i][...]
            w_v.at[i][...] = _leaky_exp(e)
            return carry2

        lax.fori_loop(0, C, edge_w, 0, unroll=4)
        cg0.wait()

        def edge_msg(i, carry2):
            wrow = w_v.at[i][...]
            row = rows_v.at[i]
            for j in range(NHEADS):
                row[pl.ds(j * NHID, NHID)] = (
                    row[pl.ds(j * NHID, NHID)] * wrow[j])
            return carry2

        lax.fori_loop(0, C, edge_msg, 0)
        pltpu.sync_copy(rows_v, sh_out.at[dst_v], add=True)
        pltpu.sync_copy(w_v, sh_den.at[dst_v], add=True)
        return carry

    lax.fori_loop(0, NCHUNK, chunk, 0)
    plsc.subcore_barrier()

    @pl.when(sid < 10)
    def _():
        pltpu.sync_copy(sh_out.at[pl.ds(r0, 1000)],
                        out_hbm.at[cid, pl.ds(r0, 1000)])
        pltpu.sync_copy(sh_den.at[pl.ds(r0, 1000)],
                        den_hbm.at[cid, pl.ds(r0, 1000)])


def _sc1(src, dst, h1, atab_s, atab_d, z128, z16):
    mesh = plsc.VectorSubcoreMesh(core_axis_name="c", subcore_axis_name="s",
                                  num_cores=NC, num_subcores=NS)
    fn = pl.kernel(
        _sc1_body,
        out_type=[
            jax.ShapeDtypeStruct((NC, N, NFEAT), _f32),
            jax.ShapeDtypeStruct((NC, N, 2 * NHEADS), _f32),
        ],
        mesh=mesh,
        scratch_types=[
            pltpu.VMEM((C,), _i32),
            pltpu.VMEM((C,), _i32),
            pltpu.VMEM((C, 2 * NHEADS), _f32),
            pltpu.VMEM((C, 2 * NHEADS), _f32),
            pltpu.VMEM((C, 2 * NHEADS), _f32),
            pltpu.VMEM((C, NFEAT), _f32),
            pltpu.VMEM_SHARED((N, NFEAT), _f32),
            pltpu.VMEM_SHARED((N, 2 * NHEADS), _f32),
            pltpu.SemaphoreType.DMA,
            pltpu.SemaphoreType.DMA,
            pltpu.SemaphoreType.DMA,
        ],
        compiler_params=pltpu.CompilerParams(use_tc_tiling_on_sc=False),
    )
    return fn(src, dst, h1, atab_s, atab_d, z128, z16)


def _sc2_body(src_hbm, dst_hbm, h2_hbm, s2_hbm, d2_hbm, z16_hbm,
              out_hbm, den_hbm,
              src_v, dst_v, gs_v, gd_v, w_v, rows_v,
              sh_out, sh_den, sem0, sem1, sem2):
    cid = lax.axis_index("c")
    sid = lax.axis_index("s")
    wid = sid * NC + cid
    r0 = sid * 1000

    @pl.when(sid < 10)
    def _():
        pltpu.sync_copy(z16_hbm.at[pl.ds(r0, 1000)],
                        sh_out.at[pl.ds(r0, 1000)])
        pltpu.sync_copy(z16_hbm.at[pl.ds(r0, 1000)],
                        sh_den.at[pl.ds(r0, 1000)])

    plsc.subcore_barrier()

    def chunk(k, carry):
        base = wid * EPW + k * C
        pltpu.sync_copy(src_hbm.at[pl.ds(base, C)], src_v)
        pltpu.sync_copy(dst_hbm.at[pl.ds(base, C)], dst_v)
        cg0 = pltpu.async_copy(h2_hbm.at[src_v], rows_v, sem0)
        cg1 = pltpu.async_copy(s2_hbm.at[src_v], gs_v, sem1)
        cg2 = pltpu.async_copy(d2_hbm.at[dst_v], gd_v, sem2)
        cg1.wait()
        cg2.wait()

        def edge_w(i, carry2):
            e = gs_v.at[i][...] + gd_v.at[i][...]
            w_v.at[i][...] = _leaky_exp(e)
            return carry2

        lax.fori_loop(0, C, edge_w, 0, unroll=4)
        cg0.wait()

        def edge_msg(i, carry2):
            row = rows_v.at[i]
            row[...] = row[...] * w_v.at[i][...]
            return carry2

        lax.fori_loop(0, C, edge_msg, 0, unroll=4)
        pltpu.sync_copy(rows_v, sh_out.at[dst_v], add=True)
        pltpu.sync_copy(w_v, sh_den.at[dst_v], add=True)
        return carry

    lax.fori_loop(0, NCHUNK, chunk, 0)
    plsc.subcore_barrier()

    @pl.when(sid < 10)
    def _():
        pltpu.sync_copy(sh_out.at[pl.ds(r0, 1000)],
                        out_hbm.at[cid, pl.ds(r0, 1000)])
        pltpu.sync_copy(sh_den.at[pl.ds(r0, 1000)],
                        den_hbm.at[cid, pl.ds(r0, 1000)])


def _sc2(src, dst, h2, s2, d2, z16):
    mesh = plsc.VectorSubcoreMesh(core_axis_name="c", subcore_axis_name="s",
                                  num_cores=NC, num_subcores=NS)
    fn = pl.kernel(
        _sc2_body,
        out_type=[
            jax.ShapeDtypeStruct((NC, N, NCLASS), _f32),
            jax.ShapeDtypeStruct((NC, N, NCLASS), _f32),
        ],
        mesh=mesh,
        scratch_types=[
            pltpu.VMEM((C,), _i32),
            pltpu.VMEM((C,), _i32),
            pltpu.VMEM((C, NCLASS), _f32),
            pltpu.VMEM((C, NCLASS), _f32),
            pltpu.VMEM((C, NCLASS), _f32),
            pltpu.VMEM((C, NCLASS), _f32),
            pltpu.VMEM_SHARED((N, NCLASS), _f32),
            pltpu.VMEM_SHARED((N, NCLASS), _f32),
            pltpu.SemaphoreType.DMA,
            pltpu.SemaphoreType.DMA,
            pltpu.SemaphoreType.DMA,
        ],
        compiler_params=pltpu.CompilerParams(use_tc_tiling_on_sc=False),
    )
    return fn(src, dst, h2, s2, d2, z16)


def kernel(x, edge_index, W1, a1, W2, a2):
    ei = edge_index.astype(_i32)
    src = ei[0]
    dst = ei[1]

    w1cat = jnp.transpose(W1, (1, 0, 2)).reshape(NFEAT, NHEADS * NHID)
    eye8 = jnp.eye(NHEADS, dtype=_f32)
    a_src = (a1[:, :NHID][..., None] * eye8[:, None, :]).reshape(NFEAT, NHEADS)
    a_dst = (a1[:, NHID:][..., None] * eye8[:, None, :]).reshape(NFEAT, NHEADS)
    rmat = jnp.kron(eye8, jnp.ones((1, NHID), dtype=_f32))
    ones16 = jnp.ones((1, NCLASS), dtype=_f32)
    a2s = a2[:NCLASS][:, None] * ones16
    a2d = a2[NCLASS:][:, None] * ones16

    z128 = jnp.zeros((N, NFEAT), _f32)
    z16 = jnp.zeros((N, 2 * NHEADS), _f32)

    h1, atab_s, atab_d = _tc1(x, w1cat, a_src, a_dst)
    acc1, den1 = _sc1(src, dst, h1, atab_s, atab_d, z128, z16)
    h2, s2, d2 = _tc2(acc1, den1, W2, a2s, a2d, rmat)
    acc2, den2 = _sc2(src, dst, h2, s2, d2, z16)
    return _tc3(acc2, den2)


# speedup vs baseline: 45.6105x; 1.2120x over previous
import functools

import jax
import jax.numpy as jnp
from jax import lax
from jax.experimental import pallas as pl
from jax.experimental.pallas import tpu as pltpu
from jax.experimental.pallas import tpu_sc as plsc

N = 10000
E = 320000
NFEAT = 128
NHID = 16
NCLASS = 16
NHEADS = 8
ALPHA = 0.2

NC, NS, L = 2, 16, 16
NW = NC * NS
EPW = E // NW
C = 80
NCHUNK = EPW // C
RPT = N // NS

_f32 = jnp.float32
_i32 = jnp.int32


def _leaky_exp(e):
    return jnp.exp(jnp.where(e >= 0, e, ALPHA * e))


def _tc1_body(x_ref, w_ref, as_ref, ad_ref, h_ref, s_ref, d_ref):
    h = jnp.dot(x_ref[...], w_ref[...], preferred_element_type=_f32)
    h_ref[...] = h
    s = jnp.dot(h, as_ref[...], preferred_element_type=_f32)
    d = jnp.dot(h, ad_ref[...], preferred_element_type=_f32)
    s_ref[...] = jnp.concatenate([s, s], axis=1)
    d_ref[...] = jnp.concatenate([d, d], axis=1)


def _tc1(x, w1cat, a_src, a_dst):
    B = 2000
    grid = (N // B,)
    return pl.pallas_call(
        _tc1_body,
        grid=grid,
        in_specs=[
            pl.BlockSpec((B, NFEAT), lambda i: (i, 0)),
            pl.BlockSpec((NFEAT, NFEAT), lambda i: (0, 0)),
            pl.BlockSpec((NFEAT, NHEADS), lambda i: (0, 0)),
            pl.BlockSpec((NFEAT, NHEADS), lambda i: (0, 0)),
        ],
        out_specs=[
            pl.BlockSpec((B, NFEAT), lambda i: (i, 0)),
            pl.BlockSpec((B, 2 * NHEADS), lambda i: (i, 0)),
            pl.BlockSpec((B, 2 * NHEADS), lambda i: (i, 0)),
        ],
        out_shape=[
            jax.ShapeDtypeStruct((N, NFEAT), _f32),
            jax.ShapeDtypeStruct((N, 2 * NHEADS), _f32),
            jax.ShapeDtypeStruct((N, 2 * NHEADS), _f32),
        ],
    )(x, w1cat, a_src, a_dst)


def _tc2_body(acc_ref, den_ref, w2_ref, a2s_ref, a2d_ref, r_ref,
              h2_ref, s2_ref, d2_ref):
    o = acc_ref[0] + acc_ref[1]
    dn = den_ref[0] + den_ref[1]
    r = 1.0 / (dn[:, :NHEADS] + 1e-9)
    db = jnp.dot(r, r_ref[...], preferred_element_type=_f32)
    on = o * db
    el = jnp.where(on > 0, on, jnp.exp(jnp.minimum(on, 0.0)) - 1.0)
    h2 = jnp.dot(el, w2_ref[...], preferred_element_type=_f32)
    h2_ref[...] = h2
    s2_ref[...] = jnp.dot(h2, a2s_ref[...], preferred_element_type=_f32)
    d2_ref[...] = jnp.dot(h2, a2d_ref[...], preferred_element_type=_f32)


def _tc2(acc1, den1, w2, a2s, a2d, rmat):
    B = 2000
    grid = (N // B,)
    return pl.pallas_call(
        _tc2_body,
        grid=grid,
        in_specs=[
            pl.BlockSpec((NC, B, NFEAT), lambda i: (0, i, 0)),
            pl.BlockSpec((NC, B, 2 * NHEADS), lambda i: (0, i, 0)),
            pl.BlockSpec((NFEAT, NCLASS), lambda i: (0, 0)),
            pl.BlockSpec((NCLASS, NCLASS), lambda i: (0, 0)),
            pl.BlockSpec((NCLASS, NCLASS), lambda i: (0, 0)),
            pl.BlockSpec((NHEADS, NFEAT), lambda i: (0, 0)),
        ],
        out_specs=[
            pl.BlockSpec((B, NCLASS), lambda i: (i, 0)),
            pl.BlockSpec((B, NCLASS), lambda i: (i, 0)),
            pl.BlockSpec((B, NCLASS), lambda i: (i, 0)),
        ],
        out_shape=[
            jax.ShapeDtypeStruct((N, NCLASS), _f32),
            jax.ShapeDtypeStruct((N, NCLASS), _f32),
            jax.ShapeDtypeStruct((N, NCLASS), _f32),
        ],
    )(acc1, den1, w2, a2s, a2d, rmat)


def _tc3_body(acc_ref, den_ref, out_ref):
    o = acc_ref[0] + acc_ref[1]
    dn = den_ref[0][:, 0:1] + den_ref[1][:, 0:1] + 1e-9
    out_ref[...] = o / dn


def _tc3(acc2, den2):
    B = 2000
    grid = (N // B,)
    return pl.pallas_call(
        _tc3_body,
        grid=grid,
        in_specs=[
            pl.BlockSpec((NC, B, NCLASS), lambda i: (0, i, 0)),
            pl.BlockSpec((NC, B, NCLASS), lambda i: (0, i, 0)),
        ],
        out_specs=pl.BlockSpec((B, NCLASS), lambda i: (i, 0)),
        out_shape=jax.ShapeDtypeStruct((N, NCLASS), _f32),
    )(acc2, den2)


def _sc1_body(src_hbm, dst_hbm, h_hbm, as_hbm, ad_hbm, z128_hbm, z16_hbm,
              out_hbm, den_hbm,
              src0, src1, dst0, dst1, gs0, gs1, gd0, gd1, rw0, rw1, w_v,
              sh_out, sh_den, semr0, semr1, sema0, sema1):
    cid = lax.axis_index("c")
    sid = lax.axis_index("s")
    wid = sid * NC + cid
    r0 = sid * 1000
    SRC, DST = (src0, src1), (dst0, dst1)
    GS, GD, ROWS = (gs0, gs1), (gd0, gd1), (rw0, rw1)
    SEMR, SEMA = (semr0, semr1), (sema0, sema1)

    @pl.when(sid < 10)
    def _():
        pltpu.sync_copy(z128_hbm.at[pl.ds(r0, 1000)],
                        sh_out.at[pl.ds(r0, 1000)])
        pltpu.sync_copy(z16_hbm.at[pl.ds(r0, 1000)],
                        sh_den.at[pl.ds(r0, 1000)])

    plsc.subcore_barrier()

    def issue(kk, b):
        base = wid * EPW + kk * C
        pltpu.sync_copy(src_hbm.at[pl.ds(base, C)], SRC[b])
        pltpu.sync_copy(dst_hbm.at[pl.ds(base, C)], DST[b])
        pltpu.async_copy(h_hbm.at[SRC[b]], ROWS[b], SEMR[b])
        pltpu.async_copy(as_hbm.at[SRC[b]], GS[b], SEMA[b])
        pltpu.async_copy(ad_hbm.at[DST[b]], GD[b], SEMA[b])

    def process(b):
        pltpu.make_async_copy(as_hbm.at[SRC[b]], GS[b], SEMA[b]).wait()
        pltpu.make_async_copy(ad_hbm.at[DST[b]], GD[b], SEMA[b]).wait()

        def edge_w(i, carry2):
            e = GS[b].at[i][...] + GD[b].at[i][...]
            w_v.at[i][...] = _leaky_exp(e)
            return carry2

        lax.fori_loop(0, C, edge_w, 0, unroll=4)
        pltpu.make_async_copy(h_hbm.at[SRC[b]], ROWS[b], SEMR[b]).wait()

        def edge_msg(i, carry2):
            wrow = w_v.at[i][...]
            row = ROWS[b].at[i]
            for j in range(NHEADS):
                row[pl.ds(j * NHID, NHID)] = (
                    row[pl.ds(j * NHID, NHID)] * wrow[j])
            return carry2

        lax.fori_loop(0, C, edge_msg, 0)
        pltpu.sync_copy(ROWS[b], sh_out.at[DST[b]], add=True)
        pltpu.sync_copy(w_v, sh_den.at[DST[b]], add=True)

    issue(0, 0)

    @pl.loop(0, NCHUNK, step=2)
    def _(k):
        for b in range(2):
            kk = k + b

            @pl.when(kk + 1 < NCHUNK)
            def _():
                issue(kk + 1, 1 - b)

            @pl.when(kk < NCHUNK)
            def _():
                process(b)

    plsc.subcore_barrier()

    @pl.when(sid < 10)
    def _():
        pltpu.sync_copy(sh_out.at[pl.ds(r0, 1000)],
                        out_hbm.at[cid, pl.ds(r0, 1000)])
        pltpu.sync_copy(sh_den.at[pl.ds(r0, 1000)],
                        den_hbm.at[cid, pl.ds(r0, 1000)])


def _sc1(src, dst, h1, atab_s, atab_d, z128, z16):
    mesh = plsc.VectorSubcoreMesh(core_axis_name="c", subcore_axis_name="s",
                                  num_cores=NC, num_subcores=NS)
    fn = pl.kernel(
        _sc1_body,
        out_type=[
            jax.ShapeDtypeStruct((NC, N, NFEAT), _f32),
            jax.ShapeDtypeStruct((NC, N, 2 * NHEADS), _f32),
        ],
        mesh=mesh,
        scratch_types=[
            pltpu.VMEM((C,), _i32),
            pltpu.VMEM((C,), _i32),
            pltpu.VMEM((C,), _i32),
            pltpu.VMEM((C,), _i32),
            pltpu.VMEM((C, 2 * NHEADS), _f32),
            pltpu.VMEM((C, 2 * NHEADS), _f32),
            pltpu.VMEM((C, 2 * NHEADS), _f32),
            pltpu.VMEM((C, 2 * NHEADS), _f32),
            pltpu.VMEM((C, NFEAT), _f32),
            pltpu.VMEM((C, NFEAT), _f32),
            pltpu.VMEM((C, 2 * NHEADS), _f32),
            pltpu.VMEM_SHARED((N, NFEAT), _f32),
            pltpu.VMEM_SHARED((N, 2 * NHEADS), _f32),
            pltpu.SemaphoreType.DMA,
            pltpu.SemaphoreType.DMA,
            pltpu.SemaphoreType.DMA,
            pltpu.SemaphoreType.DMA,
        ],
        compiler_params=pltpu.CompilerParams(use_tc_tiling_on_sc=False),
    )
    return fn(src, dst, h1, atab_s, atab_d, z128, z16)


def _sc2_body(src_hbm, dst_hbm, h2_hbm, s2_hbm, d2_hbm, z16_hbm,
              out_hbm, den_hbm,
              src0, src1, dst0, dst1, gs0, gs1, gd0, gd1, rw0, rw1, w_v,
              sh_out, sh_den, semr0, semr1, sema0, sema1):
    cid = lax.axis_index("c")
    sid = lax.axis_index("s")
    wid = sid * NC + cid
    r0 = sid * 1000
    SRC, DST = (src0, src1), (dst0, dst1)
    GS, GD, ROWS = (gs0, gs1), (gd0, gd1), (rw0, rw1)
    SEMR, SEMA = (semr0, semr1), (sema0, sema1)

    @pl.when(sid < 10)
    def _():
        pltpu.sync_copy(z16_hbm.at[pl.ds(r0, 1000)],
                        sh_out.at[pl.ds(r0, 1000)])
        pltpu.sync_copy(z16_hbm.at[pl.ds(r0, 1000)],
                        sh_den.at[pl.ds(r0, 1000)])

    plsc.subcore_barrier()

    def issue(kk, b):
        base = wid * EPW + kk * C
        pltpu.sync_copy(src_hbm.at[pl.ds(base, C)], SRC[b])
        pltpu.sync_copy(dst_hbm.at[pl.ds(base, C)], DST[b])
        pltpu.async_copy(h2_hbm.at[SRC[b]], ROWS[b], SEMR[b])
        pltpu.async_copy(s2_hbm.at[SRC[b]], GS[b], SEMA[b])
        pltpu.async_copy(d2_hbm.at[DST[b]], GD[b], SEMA[b])

    def process(b):
        pltpu.make_async_copy(s2_hbm.at[SRC[b]], GS[b], SEMA[b]).wait()
        pltpu.make_async_copy(d2_hbm.at[DST[b]], GD[b], SEMA[b]).wait()

        def edge_w(i, carry2):
            e = GS[b].at[i][...] + GD[b].at[i][...]
            w_v.at[i][...] = _leaky_exp(e)
            return carry2

        lax.fori_loop(0, C, edge_w, 0, unroll=4)
        pltpu.make_async_copy(h2_hbm.at[SRC[b]], ROWS[b], SEMR[b]).wait()

        def edge_msg(i, carry2):
            row = ROWS[b].at[i]
            row[...] = row[...] * w_v.at[i][...]
            return carry2

        lax.fori_loop(0, C, edge_msg, 0, unroll=4)
        pltpu.sync_copy(ROWS[b], sh_out.at[DST[b]], add=True)
        pltpu.sync_copy(w_v, sh_den.at[DST[b]], add=True)

    issue(0, 0)

    @pl.loop(0, NCHUNK, step=2)
    def _(k):
        for b in range(2):
            kk = k + b

            @pl.when(kk + 1 < NCHUNK)
            def _():
                issue(kk + 1, 1 - b)

            @pl.when(kk < NCHUNK)
            def _():
                process(b)

    plsc.subcore_barrier()

    @pl.when(sid < 10)
    def _():
        pltpu.sync_copy(sh_out.at[pl.ds(r0, 1000)],
                        out_hbm.at[cid, pl.ds(r0, 1000)])
        pltpu.sync_copy(sh_den.at[pl.ds(r0, 1000)],
                        den_hbm.at[cid, pl.ds(r0, 1000)])


def _sc2(src, dst, h2, s2, d2, z16):
    mesh = plsc.VectorSubcoreMesh(core_axis_name="c", subcore_axis_name="s",
                                  num_cores=NC, num_subcores=NS)
    fn = pl.kernel(
        _sc2_body,
        out_type=[
            jax.ShapeDtypeStruct((NC, N, NCLASS), _f32),
            jax.ShapeDtypeStruct((NC, N, NCLASS), _f32),
        ],
        mesh=mesh,
        scratch_types=[
            pltpu.VMEM((C,), _i32),
            pltpu.VMEM((C,), _i32),
            pltpu.VMEM((C,), _i32),
            pltpu.VMEM((C,), _i32),
            pltpu.VMEM((C, NCLASS), _f32),
            pltpu.VMEM((C, NCLASS), _f32),
            pltpu.VMEM((C, NCLASS), _f32),
            pltpu.VMEM((C, NCLASS), _f32),
            pltpu.VMEM((C, NCLASS), _f32),
            pltpu.VMEM((C, NCLASS), _f32),
            pltpu.VMEM((C, NCLASS), _f32),
            pltpu.VMEM_SHARED((N, NCLASS), _f32),
            pltpu.VMEM_SHARED((N, NCLASS), _f32),
            pltpu.SemaphoreType.DMA,
            pltpu.SemaphoreType.DMA,
            pltpu.SemaphoreType.DMA,
            pltpu.SemaphoreType.DMA,
        ],
        compiler_params=pltpu.CompilerParams(use_tc_tiling_on_sc=False),
    )
    return fn(src, dst, h2, s2, d2, z16)


def kernel(x, edge_index, W1, a1, W2, a2):
    ei = edge_index.astype(_i32)
    src = ei[0]
    dst = ei[1]

    w1cat = jnp.transpose(W1, (1, 0, 2)).reshape(NFEAT, NHEADS * NHID)
    eye8 = jnp.eye(NHEADS, dtype=_f32)
    a_src = (a1[:, :NHID][..., None] * eye8[:, None, :]).reshape(NFEAT, NHEADS)
    a_dst = (a1[:, NHID:][..., None] * eye8[:, None, :]).reshape(NFEAT, NHEADS)
    rmat = jnp.kron(eye8, jnp.ones((1, NHID), dtype=_f32))
    ones16 = jnp.ones((1, NCLASS), dtype=_f32)
    a2s = a2[:NCLASS][:, None] * ones16
    a2d = a2[NCLASS:][:, None] * ones16

    z128 = jnp.zeros((N, NFEAT), _f32)
    z16 = jnp.zeros((N, 2 * NHEADS), _f32)

    h1, atab_s, atab_d = _tc1(x, w1cat, a_src, a_dst)
    acc1, den1 = _sc1(src, dst, h1, atab_s, atab_d, z128, z16)
    h2, s2, d2 = _tc2(acc1, den1, W2, a2s, a2d, rmat)
    acc2, den2 = _sc2(src, dst, h2, s2, d2, z16)
    return _tc3(acc2, den2)


# speedup vs baseline: 51.4665x; 1.1284x over previous
import jax
import jax.numpy as jnp
from jax import lax
from jax.experimental import pallas as pl
from jax.experimental.pallas import tpu as pltpu
from jax.experimental.pallas import tpu_sc as plsc

N = 10000
E = 320000
NFEAT = 128
NHID = 16
NCLASS = 16
NHEADS = 8
ALPHA = 0.2

NC, NS, L = 2, 16, 16
NW = NC * NS
CH = 80
NCHUNK = E // CH // NW
W1COLS = NFEAT + NHID
W2COLS = 2 * NCLASS

_f32 = jnp.float32
_i32 = jnp.int32


def _leaky_exp(e):
    return jnp.exp(jnp.where(e >= 0, e, ALPHA * e))


def _tc1_body(x_ref, w_ref, as_ref, ad_ref, hs_ref, d_ref):
    h = jnp.dot(x_ref[...], w_ref[...], preferred_element_type=_f32)
    s = jnp.dot(h, as_ref[...], preferred_element_type=_f32)
    d = jnp.dot(h, ad_ref[...], preferred_element_type=_f32)
    hs_ref[...] = jnp.concatenate([h, s, s], axis=1)
    d_ref[...] = jnp.concatenate([d, d], axis=1)


def _tc1(x, w1cat, a_src, a_dst):
    B = 2000
    grid = (N // B,)
    return pl.pallas_call(
        _tc1_body,
        grid=grid,
        in_specs=[
            pl.BlockSpec((B, NFEAT), lambda i: (i, 0)),
            pl.BlockSpec((NFEAT, NFEAT), lambda i: (0, 0)),
            pl.BlockSpec((NFEAT, NHEADS), lambda i: (0, 0)),
            pl.BlockSpec((NFEAT, NHEADS), lambda i: (0, 0)),
        ],
        out_specs=[
            pl.BlockSpec((B, W1COLS), lambda i: (i, 0)),
            pl.BlockSpec((B, 2 * NHEADS), lambda i: (i, 0)),
        ],
        out_shape=[
            jax.ShapeDtypeStruct((N, W1COLS), _f32),
            jax.ShapeDtypeStruct((N, 2 * NHEADS), _f32),
        ],
    )(x, w1cat, a_src, a_dst)


def _tc2_body(acc_ref, w2_ref, a2s_ref, a2d_ref, r_ref, hs_ref, d_ref):
    o = acc_ref[0] + acc_ref[1]
    r = 1.0 / (o[:, NFEAT:NFEAT + NHEADS] + 1e-9)
    db = jnp.dot(r, r_ref[...], preferred_element_type=_f32)
    on = o[:, :NFEAT] * db
    el = jnp.where(on > 0, on, jnp.exp(jnp.minimum(on, 0.0)) - 1.0)
    h2 = jnp.dot(el, w2_ref[...], preferred_element_type=_f32)
    s2 = jnp.dot(h2, a2s_ref[...], preferred_element_type=_f32)
    d_ref[...] = jnp.dot(h2, a2d_ref[...], preferred_element_type=_f32)
    hs_ref[...] = jnp.concatenate([h2, s2], axis=1)


def _tc2(acc1, w2, a2s, a2d, rmat):
    B = 2000
    grid = (N // B,)
    return pl.pallas_call(
        _tc2_body,
        grid=grid,
        in_specs=[
            pl.BlockSpec((NC, B, W1COLS), lambda i: (0, i, 0)),
            pl.BlockSpec((NFEAT, NCLASS), lambda i: (0, 0)),
            pl.BlockSpec((NCLASS, NCLASS), lambda i: (0, 0)),
            pl.BlockSpec((NCLASS, NCLASS), lambda i: (0, 0)),
            pl.BlockSpec((NHEADS, NFEAT), lambda i: (0, 0)),
        ],
        out_specs=[
            pl.BlockSpec((B, W2COLS), lambda i: (i, 0)),
            pl.BlockSpec((B, NCLASS), lambda i: (i, 0)),
        ],
        out_shape=[
            jax.ShapeDtypeStruct((N, W2COLS), _f32),
            jax.ShapeDtypeStruct((N, NCLASS), _f32),
        ],
    )(acc1, w2, a2s, a2d, rmat)


def _tc3_body(acc_ref, out_ref):
    o = acc_ref[0] + acc_ref[1]
    out_ref[...] = o[:, :NCLASS] / (o[:, NCLASS:NCLASS + 1] + 1e-9)


def _tc3(acc2):
    B = 2000
    grid = (N // B,)
    return pl.pallas_call(
        _tc3_body,
        grid=grid,
        in_specs=[pl.BlockSpec((NC, B, W2COLS), lambda i: (0, i, 0))],
        out_specs=pl.BlockSpec((B, NCLASS), lambda i: (i, 0)),
        out_shape=jax.ShapeDtypeStruct((N, NCLASS), _f32),
    )(acc2)


def _make_sc_body(F, COLS, scale_heads):
    def body(src_hbm, dst_hbm, hs_hbm, ad_hbm, z_hbm,
             acc_hbm,
             src0, src1, dst0, dst1, rw0, rw1, gd0, gd1,
             sh_acc, semr0, semr1, sema0, sema1, sems0, sems1):
        cid = lax.axis_index("c")
        sid = lax.axis_index("s")
        wid = sid * NC + cid
        r0 = sid * 1000
        SRC, DST = (src0, src1), (dst0, dst1)
        ROWS, GD = (rw0, rw1), (gd0, gd1)
        SEMR, SEMA, SEMS = (semr0, semr1), (sema0, sema1), (sems0, sems1)

        @pl.when(sid < 10)
        def _():
            pltpu.sync_copy(z_hbm.at[pl.ds(r0, 1000)],
                            sh_acc.at[pl.ds(r0, 1000)])

        plsc.subcore_barrier()

        def issue(kk, b):
            @pl.when(kk >= 2)
            def _():
                pltpu.make_async_copy(ROWS[b], sh_acc.at[DST[b]],
                                      SEMS[b]).wait()

            base = (wid * NCHUNK + kk) * CH
            pltpu.sync_copy(src_hbm.at[pl.ds(base, CH)], SRC[b])
            pltpu.sync_copy(dst_hbm.at[pl.ds(base, CH)], DST[b])
            pltpu.async_copy(hs_hbm.at[SRC[b]], ROWS[b], SEMR[b])
            pltpu.async_copy(ad_hbm.at[DST[b]], GD[b], SEMA[b])

        def process(b):
            pltpu.make_async_copy(ad_hbm.at[DST[b]], GD[b], SEMA[b]).wait()
            pltpu.make_async_copy(hs_hbm.at[SRC[b]], ROWS[b], SEMR[b]).wait()

            def edge(i, carry):
                row = ROWS[b].at[i]
                e = row[pl.ds(F, L)] + GD[b].at[i][...]
                w = _leaky_exp(e)
                row[pl.ds(F, L)] = w
                if scale_heads:
                    for j in range(NHEADS):
                        row[pl.ds(j * NHID, NHID)] = (
                            row[pl.ds(j * NHID, NHID)] * w[j])
                else:
                    row[pl.ds(0, L)] = row[pl.ds(0, L)] * w
                return carry

            lax.fori_loop(0, CH, edge, 0)
            pltpu.async_copy(ROWS[b], sh_acc.at[DST[b]], SEMS[b], add=True)

        issue(0, 0)

        @pl.loop(0, NCHUNK + 1, step=2)
        def _(k):
            for b in range(2):
                kk = k + b

                @pl.when(kk + 1 < NCHUNK)
                def _():
                    issue(kk + 1, 1 - b)

                @pl.when(kk < NCHUNK)
                def _():
                    process(b)

        pltpu.make_async_copy(ROWS[0], sh_acc.at[DST[0]], SEMS[0]).wait()
        pltpu.make_async_copy(ROWS[1], sh_acc.at[DST[1]], SEMS[1]).wait()

        plsc.subcore_barrier()

        @pl.when(sid < 10)
        def _():
            pltpu.sync_copy(sh_acc.at[pl.ds(r0, 1000)],
                            acc_hbm.at[cid, pl.ds(r0, 1000)])

    return body


def _sc_layer(F, COLS, scale_heads, src, dst, hs, ad, z):
    mesh = plsc.VectorSubcoreMesh(core_axis_name="c", subcore_axis_name="s",
                                  num_cores=NC, num_subcores=NS)
    fn = pl.kernel(
        _make_sc_body(F, COLS, scale_heads),
        out_type=jax.ShapeDtypeStruct((NC, N, COLS), _f32),
        mesh=mesh,
        scratch_types=[
            pltpu.VMEM((CH,), _i32),
            pltpu.VMEM((CH,), _i32),
            pltpu.VMEM((CH,), _i32),
            pltpu.VMEM((CH,), _i32),
            pltpu.VMEM((CH, COLS), _f32),
            pltpu.VMEM((CH, COLS), _f32),
            pltpu.VMEM((CH, 2 * NHEADS), _f32),
            pltpu.VMEM((CH, 2 * NHEADS), _f32),
            pltpu.VMEM_SHARED((N, COLS), _f32),
            pltpu.SemaphoreType.DMA,
            pltpu.SemaphoreType.DMA,
            pltpu.SemaphoreType.DMA,
            pltpu.SemaphoreType.DMA,
            pltpu.SemaphoreType.DMA,
            pltpu.SemaphoreType.DMA,
        ],
        compiler_params=pltpu.CompilerParams(use_tc_tiling_on_sc=False),
    )
    return fn(src, dst, hs, ad, z)


def kernel(x, edge_index, W1, a1, W2, a2):
    ei = edge_index.astype(_i32)
    src = ei[0]
    dst = ei[1]

    w1cat = jnp.transpose(W1, (1, 0, 2)).reshape(NFEAT, NHEADS * NHID)
    eye8 = jnp.eye(NHEADS, dtype=_f32)
    a_src = (a1[:, :NHID][..., None] * eye8[:, None, :]).reshape(NFEAT, NHEADS)
    a_dst = (a1[:, NHID:][..., None] * eye8[:, None, :]).reshape(NFEAT, NHEADS)
    rmat = jnp.kron(eye8, jnp.ones((1, NHID), dtype=_f32))
    ones16 = jnp.ones((1, NCLASS), dtype=_f32)
    a2s = a2[:NCLASS][:, None] * ones16
    a2d = a2[NCLASS:][:, None] * ones16

    z1 = jnp.zeros((N, W1COLS), _f32)
    z2 = jnp.zeros((N, W2COLS), _f32)

    hs1, ad1 = _tc1(x, w1cat, a_src, a_dst)
    acc1 = _sc_layer(NFEAT, W1COLS, True, src, dst, hs1, ad1, z1)
    hs2, ad2 = _tc2(acc1, W2, a2s, a2d, rmat)
    acc2 = _sc_layer(NCLASS, W2COLS, False, src, dst, hs2, ad2, z2)
    return _tc3(acc2)


# speedup vs baseline: 76.1913x; 1.4804x over previous
import jax
import jax.numpy as jnp
from jax import lax
from jax.experimental import pallas as pl
from jax.experimental.pallas import tpu as pltpu
from jax.experimental.pallas import tpu_sc as plsc

N = 10000
E = 320000
NFEAT = 128
NHID = 16
NCLASS = 16
NHEADS = 8
ALPHA = 0.2

NC, NS, L = 2, 16, 16
NW = NC * NS
CH = 80
NCHUNK = E // CH // NW
W1COLS = NFEAT + NHID
W2COLS = 2 * NCLASS

_f32 = jnp.float32
_i32 = jnp.int32


def _leaky_exp(e):
    return jnp.exp(jnp.where(e >= 0, e, ALPHA * e))


def _tc1_body(x_ref, w_ref, as_ref, ad_ref, hs_ref, d_ref):
    h = jnp.dot(x_ref[...], w_ref[...], preferred_element_type=_f32)
    s = jnp.dot(h, as_ref[...], preferred_element_type=_f32)
    d = jnp.dot(h, ad_ref[...], preferred_element_type=_f32)
    hs_ref[...] = jnp.concatenate([h, s, s], axis=1)
    d_ref[...] = jnp.concatenate([d, d], axis=1)


def _tc1(x, w1cat, a_src, a_dst):
    B = 2000
    grid = (N // B,)
    return pl.pallas_call(
        _tc1_body,
        grid=grid,
        in_specs=[
            pl.BlockSpec((B, NFEAT), lambda i: (i, 0)),
            pl.BlockSpec((NFEAT, NFEAT), lambda i: (0, 0)),
            pl.BlockSpec((NFEAT, NHEADS), lambda i: (0, 0)),
            pl.BlockSpec((NFEAT, NHEADS), lambda i: (0, 0)),
        ],
        out_specs=[
            pl.BlockSpec((B, W1COLS), lambda i: (i, 0)),
            pl.BlockSpec((B, 2 * NHEADS), lambda i: (i, 0)),
        ],
        out_shape=[
            jax.ShapeDtypeStruct((N, W1COLS), _f32),
            jax.ShapeDtypeStruct((N, 2 * NHEADS), _f32),
        ],
    )(x, w1cat, a_src, a_dst)


def _tc2_body(acc_ref, w2_ref, a2s_ref, a2d_ref, r_ref, hs_ref, d_ref):
    o = acc_ref[0] + acc_ref[1]
    r = 1.0 / (o[:, NFEAT:NFEAT + NHEADS] + 1e-9)
    db = jnp.dot(r, r_ref[...], preferred_element_type=_f32)
    on = o[:, :NFEAT] * db
    el = jnp.where(on > 0, on, jnp.exp(jnp.minimum(on, 0.0)) - 1.0)
    h2 = jnp.dot(el, w2_ref[...], preferred_element_type=_f32)
    s2 = jnp.dot(h2, a2s_ref[...], preferred_element_type=_f32)
    d_ref[...] = jnp.dot(h2, a2d_ref[...], preferred_element_type=_f32)
    hs_ref[...] = jnp.concatenate([h2, s2], axis=1)


def _tc2(acc1, w2, a2s, a2d, rmat):
    B = 2000
    grid = (N // B,)
    return pl.pallas_call(
        _tc2_body,
        grid=grid,
        in_specs=[
            pl.BlockSpec((NC, B, W1COLS), lambda i: (0, i, 0)),
            pl.BlockSpec((NFEAT, NCLASS), lambda i: (0, 0)),
            pl.BlockSpec((NCLASS, NCLASS), lambda i: (0, 0)),
            pl.BlockSpec((NCLASS, NCLASS), lambda i: (0, 0)),
            pl.BlockSpec((NHEADS, NFEAT), lambda i: (0, 0)),
        ],
        out_specs=[
            pl.BlockSpec((B, W2COLS), lambda i: (i, 0)),
            pl.BlockSpec((B, NCLASS), lambda i: (i, 0)),
        ],
        out_shape=[
            jax.ShapeDtypeStruct((N, W2COLS), _f32),
            jax.ShapeDtypeStruct((N, NCLASS), _f32),
        ],
    )(acc1, w2, a2s, a2d, rmat)


def _tc3_body(acc_ref, out_ref):
    o = acc_ref[0] + acc_ref[1]
    out_ref[...] = o[:, :NCLASS] / (o[:, NCLASS:NCLASS + 1] + 1e-9)


def _tc3(acc2):
    B = 2000
    grid = (N // B,)
    return pl.pallas_call(
        _tc3_body,
        grid=grid,
        in_specs=[pl.BlockSpec((NC, B, W2COLS), lambda i: (0, i, 0))],
        out_specs=pl.BlockSpec((B, NCLASS), lambda i: (i, 0)),
        out_shape=jax.ShapeDtypeStruct((N, NCLASS), _f32),
    )(acc2)


def _make_sc_body(F, COLS, scale_heads):
    def body(src_hbm, dst_hbm, hs_hbm, ad_hbm, z_hbm,
             acc_hbm,
             src0, src1, dst0, dst1, rw0, rw1, gd0, gd1,
             sh_acc, semr0, semr1, sema0, sema1, sems0, sems1):
        cid = lax.axis_index("c")
        sid = lax.axis_index("s")
        wid = sid * NC + cid
        r0 = sid * 1000
        SRC, DST = (src0, src1), (dst0, dst1)
        ROWS, GD = (rw0, rw1), (gd0, gd1)
        SEMR, SEMA, SEMS = (semr0, semr1), (sema0, sema1), (sems0, sems1)

        @pl.when(sid < 10)
        def _():
            pltpu.sync_copy(z_hbm.at[pl.ds(r0, 1000)],
                            sh_acc.at[pl.ds(r0, 1000)])

        plsc.subcore_barrier()

        def issue(kk, b):
            @pl.when(kk >= 2)
            def _():
                pltpu.make_async_copy(ROWS[b], sh_acc.at[DST[b]],
                                      SEMS[b]).wait()

            base = (wid * NCHUNK + kk) * CH
            pltpu.sync_copy(src_hbm.at[pl.ds(base, CH)], SRC[b])
            pltpu.sync_copy(dst_hbm.at[pl.ds(base, CH)], DST[b])
            pltpu.async_copy(hs_hbm.at[SRC[b]], ROWS[b], SEMR[b])
            pltpu.async_copy(ad_hbm.at[DST[b]], GD[b], SEMA[b])

        def process(b):
            pltpu.make_async_copy(ad_hbm.at[DST[b]], GD[b], SEMA[b]).wait()
            pltpu.make_async_copy(hs_hbm.at[SRC[b]], ROWS[b], SEMR[b]).wait()

            @plsc.parallel_loop(0, CH, unroll=(2 if scale_heads else 4))
            def _(i):
                row = ROWS[b].at[i]
                e = row[pl.ds(F, L)] + GD[b].at[i][...]
                w = _leaky_exp(e)
                row[pl.ds(F, L)] = w
                if scale_heads:
                    for j in range(NHEADS):
                        row[pl.ds(j * NHID, NHID)] = (
                            row[pl.ds(j * NHID, NHID)] * w[j])
                else:
                    row[pl.ds(0, L)] = row[pl.ds(0, L)] * w
            pltpu.async_copy(ROWS[b], sh_acc.at[DST[b]], SEMS[b], add=True)

        issue(0, 0)

        @pl.loop(0, NCHUNK + 1, step=2)
        def _(k):
            for b in range(2):
                kk = k + b

                @pl.when(kk + 1 < NCHUNK)
                def _():
                    issue(kk + 1, 1 - b)

                @pl.when(kk < NCHUNK)
                def _():
                    process(b)

        pltpu.make_async_copy(ROWS[0], sh_acc.at[DST[0]], SEMS[0]).wait()
        pltpu.make_async_copy(ROWS[1], sh_acc.at[DST[1]], SEMS[1]).wait()

        plsc.subcore_barrier()

        @pl.when(sid < 10)
        def _():
            pltpu.sync_copy(sh_acc.at[pl.ds(r0, 1000)],
                            acc_hbm.at[cid, pl.ds(r0, 1000)])

    return body


def _sc_layer(F, COLS, scale_heads, src, dst, hs, ad, z):
    mesh = plsc.VectorSubcoreMesh(core_axis_name="c", subcore_axis_name="s",
                                  num_cores=NC, num_subcores=NS)
    fn = pl.kernel(
        _make_sc_body(F, COLS, scale_heads),
        out_type=jax.ShapeDtypeStruct((NC, N, COLS), _f32),
        mesh=mesh,
        scratch_types=[
            pltpu.VMEM((CH,), _i32),
            pltpu.VMEM((CH,), _i32),
            pltpu.VMEM((CH,), _i32),
            pltpu.VMEM((CH,), _i32),
            pltpu.VMEM((CH, COLS), _f32),
            pltpu.VMEM((CH, COLS), _f32),
            pltpu.VMEM((CH, 2 * NHEADS), _f32),
            pltpu.VMEM((CH, 2 * NHEADS), _f32),
            pltpu.VMEM_SHARED((N, COLS), _f32),
            pltpu.SemaphoreType.DMA,
            pltpu.SemaphoreType.DMA,
            pltpu.SemaphoreType.DMA,
            pltpu.SemaphoreType.DMA,
            pltpu.SemaphoreType.DMA,
            pltpu.SemaphoreType.DMA,
        ],
        compiler_params=pltpu.CompilerParams(use_tc_tiling_on_sc=False),
    )
    return fn(src, dst, hs, ad, z)


def kernel(x, edge_index, W1, a1, W2, a2):
    ei = edge_index.astype(_i32)
    src = ei[0]
    dst = ei[1]

    w1cat = jnp.transpose(W1, (1, 0, 2)).reshape(NFEAT, NHEADS * NHID)
    eye8 = jnp.eye(NHEADS, dtype=_f32)
    a_src = (a1[:, :NHID][..., None] * eye8[:, None, :]).reshape(NFEAT, NHEADS)
    a_dst = (a1[:, NHID:][..., None] * eye8[:, None, :]).reshape(NFEAT, NHEADS)
    rmat = jnp.kron(eye8, jnp.ones((1, NHID), dtype=_f32))
    ones16 = jnp.ones((1, NCLASS), dtype=_f32)
    a2s = a2[:NCLASS][:, None] * ones16
    a2d = a2[NCLASS:][:, None] * ones16

    z1 = jnp.zeros((N, W1COLS), _f32)
    z2 = jnp.zeros((N, W2COLS), _f32)

    hs1, ad1 = _tc1(x, w1cat, a_src, a_dst)
    acc1 = _sc_layer(NFEAT, W1COLS, True, src, dst, hs1, ad1, z1)
    hs2, ad2 = _tc2(acc1, W2, a2s, a2d, rmat)
    acc2 = _sc_layer(NCLASS, W2COLS, False, src, dst, hs2, ad2, z2)
    return _tc3(acc2)


# speedup vs baseline: 77.7532x; 1.0205x over previous
import jax
import jax.numpy as jnp
from jax import lax
from jax.experimental import pallas as pl
from jax.experimental.pallas import tpu as pltpu
from jax.experimental.pallas import tpu_sc as plsc

N = 10000
E = 320000
NFEAT = 128
NHID = 16
NCLASS = 16
NHEADS = 8
ALPHA = 0.2

NC, NS, L = 2, 16, 16
NW = NC * NS
CH = 80
NCHUNK = E // CH // NW
W1COLS = NFEAT + NHID
W2COLS = 2 * NCLASS

_f32 = jnp.float32
_i32 = jnp.int32


def _leaky_exp(e):
    return jnp.exp(jnp.where(e >= 0, e, ALPHA * e))


def _tc1_body(x_ref, w_ref, as_ref, ad_ref, hs_ref, d_ref):
    h = jnp.dot(x_ref[...], w_ref[...], preferred_element_type=_f32)
    s = jnp.dot(h, as_ref[...], preferred_element_type=_f32)
    d = jnp.dot(h, ad_ref[...], preferred_element_type=_f32)
    hs_ref[...] = jnp.concatenate([h, s, s], axis=1)
    d_ref[...] = jnp.concatenate([d, d], axis=1)


def _tc1(x, w1cat, a_src, a_dst):
    B = 2000
    grid = (N // B,)
    return pl.pallas_call(
        _tc1_body,
        grid=grid,
        in_specs=[
            pl.BlockSpec((B, NFEAT), lambda i: (i, 0)),
            pl.BlockSpec((NFEAT, NFEAT), lambda i: (0, 0)),
            pl.BlockSpec((NFEAT, NHEADS), lambda i: (0, 0)),
            pl.BlockSpec((NFEAT, NHEADS), lambda i: (0, 0)),
        ],
        out_specs=[
            pl.BlockSpec((B, W1COLS), lambda i: (i, 0)),
            pl.BlockSpec((B, 2 * NHEADS), lambda i: (i, 0)),
        ],
        out_shape=[
            jax.ShapeDtypeStruct((N, W1COLS), _f32),
            jax.ShapeDtypeStruct((N, 2 * NHEADS), _f32),
        ],
    )(x, w1cat, a_src, a_dst)


def _tc2_body(acc_ref, w2_ref, a2s_ref, a2d_ref, r_ref, hs_ref, d_ref):
    o = acc_ref[0] + acc_ref[1]
    r = 1.0 / (o[:, NFEAT:NFEAT + NHEADS] + 1e-9)
    db = jnp.dot(r, r_ref[...], preferred_element_type=_f32)
    on = o[:, :NFEAT] * db
    el = jnp.where(on > 0, on, jnp.exp(jnp.minimum(on, 0.0)) - 1.0)
    h2 = jnp.dot(el, w2_ref[...], preferred_element_type=_f32)
    s2 = jnp.dot(h2, a2s_ref[...], preferred_element_type=_f32)
    d_ref[...] = jnp.dot(h2, a2d_ref[...], preferred_element_type=_f32)
    hs_ref[...] = jnp.concatenate([h2, s2], axis=1)


def _tc2(acc1, w2, a2s, a2d, rmat):
    B = 2000
    grid = (N // B,)
    return pl.pallas_call(
        _tc2_body,
        grid=grid,
        in_specs=[
            pl.BlockSpec((NC, B, W1COLS), lambda i: (0, i, 0)),
            pl.BlockSpec((NFEAT, NCLASS), lambda i: (0, 0)),
            pl.BlockSpec((NCLASS, NCLASS), lambda i: (0, 0)),
            pl.BlockSpec((NCLASS, NCLASS), lambda i: (0, 0)),
            pl.BlockSpec((NHEADS, NFEAT), lambda i: (0, 0)),
        ],
        out_specs=[
            pl.BlockSpec((B, W2COLS), lambda i: (i, 0)),
            pl.BlockSpec((B, NCLASS), lambda i: (i, 0)),
        ],
        out_shape=[
            jax.ShapeDtypeStruct((N, W2COLS), _f32),
            jax.ShapeDtypeStruct((N, NCLASS), _f32),
        ],
    )(acc1, w2, a2s, a2d, rmat)


def _tc3_body(acc_ref, out_ref):
    o = acc_ref[0] + acc_ref[1]
    out_ref[...] = o[:, :NCLASS] / (o[:, NCLASS:NCLASS + 1] + 1e-9)


def _tc3(acc2):
    B = 2000
    grid = (N // B,)
    return pl.pallas_call(
        _tc3_body,
        grid=grid,
        in_specs=[pl.BlockSpec((NC, B, W2COLS), lambda i: (0, i, 0))],
        out_specs=pl.BlockSpec((B, NCLASS), lambda i: (i, 0)),
        out_shape=jax.ShapeDtypeStruct((N, NCLASS), _f32),
    )(acc2)


def _make_sc_body(F, COLS, scale_heads):
    def body(ei_hbm, hs_hbm, ad_hbm, z_hbm,
             acc_hbm,
             src0, src1, src2, dst0, dst1, dst2, rw0, rw1, rw2,
             gd0, gd1, gd2,
             sh_acc, semr0, semr1, semr2, sema0, sema1, sema2,
             sems0, sems1, sems2):
        cid = lax.axis_index("c")
        sid = lax.axis_index("s")
        wid = sid * NC + cid
        r0 = sid * 1000
        SRC, DST = (src0, src1, src2), (dst0, dst1, dst2)
        ROWS, GD = (rw0, rw1, rw2), (gd0, gd1, gd2)
        SEMR = (semr0, semr1, semr2)
        SEMA = (sema0, sema1, sema2)
        SEMS = (sems0, sems1, sems2)

        @pl.when(sid < 10)
        def _():
            pltpu.sync_copy(z_hbm.at[pl.ds(r0, 1000)],
                            sh_acc.at[pl.ds(r0, 1000)])

        plsc.subcore_barrier()

        def issue(kk, b):
            @pl.when(kk >= 3)
            def _():
                pltpu.make_async_copy(ROWS[b], sh_acc.at[DST[b]],
                                      SEMS[b]).wait()

            base = (wid * NCHUNK + kk) * CH
            pltpu.sync_copy(ei_hbm.at[0, pl.ds(base, CH)], SRC[b])
            pltpu.sync_copy(ei_hbm.at[1, pl.ds(base, CH)], DST[b])
            pltpu.async_copy(hs_hbm.at[SRC[b]], ROWS[b], SEMR[b])
            pltpu.async_copy(ad_hbm.at[DST[b]], GD[b], SEMA[b])

        def process(b):
            pltpu.make_async_copy(ad_hbm.at[DST[b]], GD[b], SEMA[b]).wait()
            pltpu.make_async_copy(hs_hbm.at[SRC[b]], ROWS[b], SEMR[b]).wait()

            @plsc.parallel_loop(0, CH, unroll=4)
            def _(i):
                row = ROWS[b].at[i]
                e = row[pl.ds(F, L)] + GD[b].at[i][...]
                w = _leaky_exp(e)
                row[pl.ds(F, L)] = w
                if scale_heads:
                    for j in range(NHEADS):
                        row[pl.ds(j * NHID, NHID)] = (
                            row[pl.ds(j * NHID, NHID)] * w[j])
                else:
                    row[pl.ds(0, L)] = row[pl.ds(0, L)] * w

            pltpu.async_copy(ROWS[b], sh_acc.at[DST[b]], SEMS[b], add=True)

        issue(0, 0)
        issue(1, 1)

        @pl.loop(0, NCHUNK + 1, step=3)
        def _(k):
            for b in range(3):
                kk = k + b

                @pl.when(kk + 2 < NCHUNK)
                def _():
                    issue(kk + 2, (b + 2) % 3)

                @pl.when(kk < NCHUNK)
                def _():
                    process(b)

        pltpu.make_async_copy(ROWS[0], sh_acc.at[DST[0]], SEMS[0]).wait()
        pltpu.make_async_copy(ROWS[1], sh_acc.at[DST[1]], SEMS[1]).wait()
        pltpu.make_async_copy(ROWS[2], sh_acc.at[DST[2]], SEMS[2]).wait()

        plsc.subcore_barrier()

        @pl.when(sid < 10)
        def _():
            pltpu.sync_copy(sh_acc.at[pl.ds(r0, 1000)],
                            acc_hbm.at[cid, pl.ds(r0, 1000)])

    return body


def _sc_layer(F, COLS, scale_heads, ei, hs, ad, z):
    mesh = plsc.VectorSubcoreMesh(core_axis_name="c", subcore_axis_name="s",
                                  num_cores=NC, num_subcores=NS)
    fn = pl.kernel(
        _make_sc_body(F, COLS, scale_heads),
        out_type=jax.ShapeDtypeStruct((NC, N, COLS), _f32),
        mesh=mesh,
        scratch_types=(
            [pltpu.VMEM((CH,), _i32)] * 6
            + [pltpu.VMEM((CH, COLS), _f32)] * 3
            + [pltpu.VMEM((CH, 2 * NHEADS), _f32)] * 3
            + [pltpu.VMEM_SHARED((N, COLS), _f32)]
            + [pltpu.SemaphoreType.DMA] * 9
        ),
        compiler_params=pltpu.CompilerParams(use_tc_tiling_on_sc=False),
    )
    return fn(ei, hs, ad, z)


def kernel(x, edge_index, W1, a1, W2, a2):
    ei = edge_index.astype(_i32)

    w1cat = jnp.transpose(W1, (1, 0, 2)).reshape(NFEAT, NHEADS * NHID)
    eye8 = jnp.eye(NHEADS, dtype=_f32)
    a_src = (a1[:, :NHID][..., None] * eye8[:, None, :]).reshape(NFEAT, NHEADS)
    a_dst = (a1[:, NHID:][..., None] * eye8[:, None, :]).reshape(NFEAT, NHEADS)
    rmat = jnp.kron(eye8, jnp.ones((1, NHID), dtype=_f32))
    ones16 = jnp.ones((1, NCLASS), dtype=_f32)
    a2s = a2[:NCLASS][:, None] * ones16
    a2d = a2[NCLASS:][:, None] * ones16

    z1 = jnp.zeros((N, W1COLS), _f32)
    z2 = jnp.zeros((N, W2COLS), _f32)

    hs1, ad1 = _tc1(x, w1cat, a_src, a_dst)
    acc1 = _sc_layer(NFEAT, W1COLS, True, ei, hs1, ad1, z1)
    hs2, ad2 = _tc2(acc1, W2, a2s, a2d, rmat)
    acc2 = _sc_layer(NCLASS, W2COLS, False, ei, hs2, ad2, z2)
    return _tc3(acc2)


# speedup vs baseline: 95.2562x; 1.2251x over previous
import jax
import jax.numpy as jnp
from jax import lax
from jax.experimental import pallas as pl
from jax.experimental.pallas import tpu as pltpu
from jax.experimental.pallas import tpu_sc as plsc

N = 10000
E = 320000
NFEAT = 128
NHID = 16
NCLASS = 16
NHEADS = 8
ALPHA = 0.2

NC, NS, L = 2, 16, 16
NW = NC * NS
CH = 80
NCHUNK = E // CH // NW
W1COLS = NFEAT + NHID
W2COLS = 2 * NCLASS

_f32 = jnp.float32
_i32 = jnp.int32


def _leaky_exp(e):
    return jnp.exp(jnp.where(e >= 0, e, ALPHA * e))


def _tc1_body(x_ref, w_ref, as_ref, ad_ref, hs_ref, d_ref):
    h = jnp.dot(x_ref[...], w_ref[...], preferred_element_type=_f32)
    s = jnp.dot(h, as_ref[...], preferred_element_type=_f32)
    d = jnp.dot(h, ad_ref[...], preferred_element_type=_f32)
    hs_ref[...] = jnp.concatenate([h, s, s], axis=1)
    d_ref[...] = jnp.concatenate([d, d], axis=1)


def _tc1(x, w1cat, a_src, a_dst):
    B = 2000
    grid = (N // B,)
    return pl.pallas_call(
        _tc1_body,
        grid=grid,
        in_specs=[
            pl.BlockSpec((B, NFEAT), lambda i: (i, 0)),
            pl.BlockSpec((NFEAT, NFEAT), lambda i: (0, 0)),
            pl.BlockSpec((NFEAT, NHEADS), lambda i: (0, 0)),
            pl.BlockSpec((NFEAT, NHEADS), lambda i: (0, 0)),
        ],
        out_specs=[
            pl.BlockSpec((B, W1COLS), lambda i: (i, 0)),
            pl.BlockSpec((B, 2 * NHEADS), lambda i: (i, 0)),
        ],
        out_shape=[
            jax.ShapeDtypeStruct((N, W1COLS), _f32),
            jax.ShapeDtypeStruct((N, 2 * NHEADS), _f32),
        ],
    )(x, w1cat, a_src, a_dst)


def _tc2_body(acc_ref, w2_ref, a2s_ref, a2d_ref, r_ref, hs_ref, d_ref):
    o = acc_ref[0] + acc_ref[1]
    r = 1.0 / (o[:, NFEAT:NFEAT + NHEADS] + 1e-9)
    db = jnp.dot(r, r_ref[...], preferred_element_type=_f32)
    on = o[:, :NFEAT] * db
    el = jnp.where(on > 0, on, jnp.exp(jnp.minimum(on, 0.0)) - 1.0)
    h2 = jnp.dot(el, w2_ref[...], preferred_element_type=_f32)
    s2 = jnp.dot(h2, a2s_ref[...], preferred_element_type=_f32)
    d_ref[...] = jnp.dot(h2, a2d_ref[...], preferred_element_type=_f32)
    hs_ref[...] = jnp.concatenate([h2, s2], axis=1)


def _tc2(acc1, w2, a2s, a2d, rmat):
    B = 2000
    grid = (N // B,)
    return pl.pallas_call(
        _tc2_body,
        grid=grid,
        in_specs=[
            pl.BlockSpec((NC, B, W1COLS), lambda i: (0, i, 0)),
            pl.BlockSpec((NFEAT, NCLASS), lambda i: (0, 0)),
            pl.BlockSpec((NCLASS, NCLASS), lambda i: (0, 0)),
            pl.BlockSpec((NCLASS, NCLASS), lambda i: (0, 0)),
            pl.BlockSpec((NHEADS, NFEAT), lambda i: (0, 0)),
        ],
        out_specs=[
            pl.BlockSpec((B, W2COLS), lambda i: (i, 0)),
            pl.BlockSpec((B, NCLASS), lambda i: (i, 0)),
        ],
        out_shape=[
            jax.ShapeDtypeStruct((N, W2COLS), _f32),
            jax.ShapeDtypeStruct((N, NCLASS), _f32),
        ],
    )(acc1, w2, a2s, a2d, rmat)


def _tc3_body(acc_ref, out_ref):
    o = acc_ref[0] + acc_ref[1]
    out_ref[...] = o[:, :NCLASS] / (o[:, NCLASS:NCLASS + 1] + 1e-9)


def _tc3(acc2):
    B = 2000
    grid = (N // B,)
    return pl.pallas_call(
        _tc3_body,
        grid=grid,
        in_specs=[pl.BlockSpec((NC, B, W2COLS), lambda i: (0, i, 0))],
        out_specs=pl.BlockSpec((B, NCLASS), lambda i: (i, 0)),
        out_shape=jax.ShapeDtypeStruct((N, NCLASS), _f32),
    )(acc2)


def _make_sc_body(F, COLS, stage_hs, scale_heads):
    def body(ei_hbm, hs_hbm, ad_hbm, z_hbm,
             acc_hbm,
             src0, src1, dst0, dst1, rw0, rw1, gd0, gd1,
             sh_acc, sh_ad, sh_hs, semr0, semr1, sema0, sema1,
             sems0, sems1, semi0, semi1):
        cid = lax.axis_index("c")
        sid = lax.axis_index("s")
        wid = sid * NC + cid
        r0 = sid * 1000
        SRC, DST = (src0, src1), (dst0, dst1)
        ROWS, GD = (rw0, rw1), (gd0, gd1)
        SEMR, SEMA = (semr0, semr1), (sema0, sema1)
        SEMS, SEMI = (sems0, sems1), (semi0, semi1)
        hs_src = sh_hs if stage_hs else hs_hbm

        @pl.when(sid < 10)
        def _():
            pltpu.sync_copy(z_hbm.at[pl.ds(r0, 1000)],
                            sh_acc.at[pl.ds(r0, 1000)])
            pltpu.sync_copy(ad_hbm.at[pl.ds(r0, 1000)],
                            sh_ad.at[pl.ds(r0, 1000)])
            if stage_hs:
                pltpu.sync_copy(hs_hbm.at[pl.ds(r0, 1000)],
                                sh_hs.at[pl.ds(r0, 1000)])

        plsc.subcore_barrier()

        def issue(kk, b):
            @pl.when(kk >= 2)
            def _():
                pltpu.make_async_copy(ROWS[b], sh_acc.at[DST[b]],
                                      SEMS[b]).wait()

            base = (wid * NCHUNK + kk) * CH
            pltpu.async_copy(ei_hbm.at[0, pl.ds(base, CH)], SRC[b], SEMI[b])
            pltpu.async_copy(ei_hbm.at[1, pl.ds(base, CH)], DST[b], SEMI[b])
            pltpu.make_async_copy(ei_hbm.at[0, pl.ds(base, CH)], SRC[b],
                                  SEMI[b]).wait()
            pltpu.make_async_copy(ei_hbm.at[1, pl.ds(base, CH)], DST[b],
                                  SEMI[b]).wait()
            pltpu.async_copy(hs_src.at[SRC[b]], ROWS[b], SEMR[b])
            pltpu.async_copy(sh_ad.at[DST[b]], GD[b], SEMA[b])

        def process(b):
            pltpu.make_async_copy(sh_ad.at[DST[b]], GD[b], SEMA[b]).wait()
            pltpu.make_async_copy(hs_src.at[SRC[b]], ROWS[b], SEMR[b]).wait()

            @plsc.parallel_loop(0, CH, unroll=4)
            def _(i):
                row = ROWS[b].at[i]
                e = row[pl.ds(F, L)] + GD[b].at[i][...]
                w = _leaky_exp(e)
                row[pl.ds(F, L)] = w
                if scale_heads:
                    for j in range(NHEADS):
                        row[pl.ds(j * NHID, NHID)] = (
                            row[pl.ds(j * NHID, NHID)] * w[j])
                else:
                    row[pl.ds(0, L)] = row[pl.ds(0, L)] * w

            pltpu.async_copy(ROWS[b], sh_acc.at[DST[b]], SEMS[b], add=True)

        issue(0, 0)

        @pl.loop(0, NCHUNK + 1, step=2)
        def _(k):
            for b in range(2):
                kk = k + b

                @pl.when(kk + 1 < NCHUNK)
                def _():
                    issue(kk + 1, 1 - b)

                @pl.when(kk < NCHUNK)
                def _():
                    process(b)

        pltpu.make_async_copy(ROWS[0], sh_acc.at[DST[0]], SEMS[0]).wait()
        pltpu.make_async_copy(ROWS[1], sh_acc.at[DST[1]], SEMS[1]).wait()

        plsc.subcore_barrier()

        @pl.when(sid < 10)
        def _():
            pltpu.sync_copy(sh_acc.at[pl.ds(r0, 1000)],
                            acc_hbm.at[cid, pl.ds(r0, 1000)])

    return body


def _sc_layer(F, COLS, stage_hs, scale_heads, ei, hs, ad, z):
    mesh = plsc.VectorSubcoreMesh(core_axis_name="c", subcore_axis_name="s",
                                  num_cores=NC, num_subcores=NS)
    fn = pl.kernel(
        _make_sc_body(F, COLS, stage_hs, scale_heads),
        out_type=jax.ShapeDtypeStruct((NC, N, COLS), _f32),
        mesh=mesh,
        scratch_types=(
            [pltpu.VMEM((CH,), _i32)] * 4
            + [pltpu.VMEM((CH, COLS), _f32)] * 2
            + [pltpu.VMEM((CH, 2 * NHEADS), _f32)] * 2
            + [pltpu.VMEM_SHARED((N, COLS), _f32)]
            + [pltpu.VMEM_SHARED((N, 2 * NHEADS), _f32)]
            + [pltpu.VMEM_SHARED((N, COLS if stage_hs else 1), _f32)]
            + [pltpu.SemaphoreType.DMA] * 8
        ),
        compiler_params=pltpu.CompilerParams(use_tc_tiling_on_sc=False),
    )
    return fn(ei, hs, ad, z)


def kernel(x, edge_index, W1, a1, W2, a2):
    ei = edge_index.astype(_i32)

    w1cat = jnp.transpose(W1, (1, 0, 2)).reshape(NFEAT, NHEADS * NHID)
    eye8 = jnp.eye(NHEADS, dtype=_f32)
    a_src = (a1[:, :NHID][..., None] * eye8[:, None, :]).reshape(NFEAT, NHEADS)
    a_dst = (a1[:, NHID:][..., None] * eye8[:, None, :]).reshape(NFEAT, NHEADS)
    rmat = jnp.kron(eye8, jnp.ones((1, NHID), dtype=_f32))
    ones16 = jnp.ones((1, NCLASS), dtype=_f32)
    a2s = a2[:NCLASS][:, None] * ones16
    a2d = a2[NCLASS:][:, None] * ones16

    z1 = jnp.zeros((N, W1COLS), _f32)
    z2 = jnp.zeros((N, W2COLS), _f32)

    hs1, ad1 = _tc1(x, w1cat, a_src, a_dst)
    acc1 = _sc_layer(NFEAT, W1COLS, False, True, ei, hs1, ad1, z1)
    hs2, ad2 = _tc2(acc1, W2, a2s, a2d, rmat)
    acc2 = _sc_layer(NCLASS, W2COLS, True, False, ei, hs2, ad2, z2)
    return _tc3(acc2)


# speedup vs baseline: 99.7785x; 1.0475x over previous
import jax
import jax.numpy as jnp
from jax import lax
from jax.experimental import pallas as pl
from jax.experimental.pallas import tpu as pltpu
from jax.experimental.pallas import tpu_sc as plsc

N = 10000
E = 320000
NFEAT = 128
NHID = 16
NCLASS = 16
NHEADS = 8
ALPHA = 0.2

NC, NS, L = 2, 16, 16
NW = NC * NS
CH1 = 80
CH2 = 128
W1COLS = NFEAT + NHID
W2COLS = 2 * NCLASS

_f32 = jnp.float32
_i32 = jnp.int32


def _leaky_exp(e):
    return jnp.exp(jnp.where(e >= 0, e, ALPHA * e))


def _tc1_body(x_ref, w_ref, as_ref, ad_ref, hs_ref, d_ref):
    h = jnp.dot(x_ref[...], w_ref[...], preferred_element_type=_f32)
    s = jnp.dot(h, as_ref[...], preferred_element_type=_f32)
    d = jnp.dot(h, ad_ref[...], preferred_element_type=_f32)
    hs_ref[...] = jnp.concatenate([h, s, s], axis=1)
    d_ref[...] = jnp.concatenate([d, d], axis=1)


def _tc1(x, w1cat, a_src, a_dst):
    B = 2000
    grid = (N // B,)
    return pl.pallas_call(
        _tc1_body,
        grid=grid,
        in_specs=[
            pl.BlockSpec((B, NFEAT), lambda i: (i, 0)),
            pl.BlockSpec((NFEAT, NFEAT), lambda i: (0, 0)),
            pl.BlockSpec((NFEAT, NHEADS), lambda i: (0, 0)),
            pl.BlockSpec((NFEAT, NHEADS), lambda i: (0, 0)),
        ],
        out_specs=[
            pl.BlockSpec((B, W1COLS), lambda i: (i, 0)),
            pl.BlockSpec((B, 2 * NHEADS), lambda i: (i, 0)),
        ],
        out_shape=[
            jax.ShapeDtypeStruct((N, W1COLS), _f32),
            jax.ShapeDtypeStruct((N, 2 * NHEADS), _f32),
        ],
    )(x, w1cat, a_src, a_dst)


def _tc2_body(acc_ref, w2_ref, a2s_ref, a2d_ref, r_ref, hs_ref, d_ref):
    o = acc_ref[0] + acc_ref[1]
    r = 1.0 / (o[:, NFEAT:NFEAT + NHEADS] + 1e-9)
    db = jnp.dot(r, r_ref[...], preferred_element_type=_f32)
    on = o[:, :NFEAT] * db
    el = jnp.where(on > 0, on, jnp.exp(jnp.minimum(on, 0.0)) - 1.0)
    h2 = jnp.dot(el, w2_ref[...], preferred_element_type=_f32)
    s2 = jnp.dot(h2, a2s_ref[...], preferred_element_type=_f32)
    d_ref[...] = jnp.dot(h2, a2d_ref[...], preferred_element_type=_f32)
    hs_ref[...] = jnp.concatenate([h2, s2], axis=1)


def _tc2(acc1, w2, a2s, a2d, rmat):
    B = 2000
    grid = (N // B,)
    return pl.pallas_call(
        _tc2_body,
        grid=grid,
        in_specs=[
            pl.BlockSpec((NC, B, W1COLS), lambda i: (0, i, 0)),
            pl.BlockSpec((NFEAT, NCLASS), lambda i: (0, 0)),
            pl.BlockSpec((NCLASS, NCLASS), lambda i: (0, 0)),
            pl.BlockSpec((NCLASS, NCLASS), lambda i: (0, 0)),
            pl.BlockSpec((NHEADS, NFEAT), lambda i: (0, 0)),
        ],
        out_specs=[
            pl.BlockSpec((B, W2COLS), lambda i: (i, 0)),
            pl.BlockSpec((B, NCLASS), lambda i: (i, 0)),
        ],
        out_shape=[
            jax.ShapeDtypeStruct((N, W2COLS), _f32),
            jax.ShapeDtypeStruct((N, NCLASS), _f32),
        ],
    )(acc1, w2, a2s, a2d, rmat)


def _tc3_body(acc_ref, out_ref):
    o = acc_ref[0] + acc_ref[1]
    out_ref[...] = o[:, :NCLASS] / (o[:, NCLASS:NCLASS + 1] + 1e-9)


def _tc3(acc2):
    B = 2000
    grid = (N // B,)
    return pl.pallas_call(
        _tc3_body,
        grid=grid,
        in_specs=[pl.BlockSpec((NC, B, W2COLS), lambda i: (0, i, 0))],
        out_specs=pl.BlockSpec((B, NCLASS), lambda i: (i, 0)),
        out_shape=jax.ShapeDtypeStruct((N, NCLASS), _f32),
    )(acc2)


def _make_sc_body(F, COLS, CH, stage_hs, scale_heads):
    def body(ei_hbm, hs_hbm, ad_hbm, z_hbm,
             acc_hbm,
             src0, src1, dst0, dst1, rw0, rw1, gd0, gd1,
             sh_acc, sh_ad, sh_hs, semr0, semr1, sema0, sema1,
             sems0, sems1, semi0, semi1):
        cid = lax.axis_index("c")
        sid = lax.axis_index("s")
        wid = sid * NC + cid
        r0 = sid * 1000
        SRC, DST = (src0, src1), (dst0, dst1)
        ROWS, GD = (rw0, rw1), (gd0, gd1)
        SEMR, SEMA = (semr0, semr1), (sema0, sema1)
        SEMS, SEMI = (sems0, sems1), (semi0, semi1)
        hs_src = sh_hs if stage_hs else hs_hbm
        totch = E // CH
        nb = totch // NW
        extra = totch - nb * NW
        nch = nb + jnp.where(wid < extra, 1, 0)
        nloop = nb + extra % 2 + (2 if extra else 0)

        @pl.when(sid < 10)
        def _():
            pltpu.sync_copy(z_hbm.at[pl.ds(r0, 1000)],
                            sh_acc.at[pl.ds(r0, 1000)])
            pltpu.sync_copy(ad_hbm.at[pl.ds(r0, 1000)],
                            sh_ad.at[pl.ds(r0, 1000)])
            if stage_hs:
                pltpu.sync_copy(hs_hbm.at[pl.ds(r0, 1000)],
                                sh_hs.at[pl.ds(r0, 1000)])

        plsc.subcore_barrier()

        def issue(kk, b):
            @pl.when(kk >= 2)
            def _():
                pltpu.make_async_copy(ROWS[b], sh_acc.at[DST[b]],
                                      SEMS[b]).wait()

            base = (wid + NW * kk) * CH
            pltpu.async_copy(ei_hbm.at[0, pl.ds(base, CH)], SRC[b], SEMI[b])
            pltpu.async_copy(ei_hbm.at[1, pl.ds(base, CH)], DST[b], SEMI[b])
            pltpu.make_async_copy(ei_hbm.at[0, pl.ds(base, CH)], SRC[b],
                                  SEMI[b]).wait()
            pltpu.make_async_copy(ei_hbm.at[1, pl.ds(base, CH)], DST[b],
                                  SEMI[b]).wait()
            pltpu.async_copy(hs_src.at[SRC[b]], ROWS[b], SEMR[b])
            pltpu.async_copy(sh_ad.at[DST[b]], GD[b], SEMA[b])

        def process(b):
            pltpu.make_async_copy(sh_ad.at[DST[b]], GD[b], SEMA[b]).wait()
            pltpu.make_async_copy(hs_src.at[SRC[b]], ROWS[b], SEMR[b]).wait()

            @plsc.parallel_loop(0, CH, unroll=4)
            def _(i):
                row = ROWS[b].at[i]
                e = row[pl.ds(F, L)] + GD[b].at[i][...]
                w = _leaky_exp(e)
                row[pl.ds(F, L)] = w
                if scale_heads:
                    for j in range(NHEADS):
                        row[pl.ds(j * NHID, NHID)] = (
                            row[pl.ds(j * NHID, NHID)] * w[j])
                else:
                    row[pl.ds(0, L)] = row[pl.ds(0, L)] * w

            pltpu.async_copy(ROWS[b], sh_acc.at[DST[b]], SEMS[b], add=True)

        issue(0, 0)

        @pl.loop(0, nloop, step=2)
        def _(k):
            for b in range(2):
                kk = k + b

                @pl.when(kk + 1 < nch)
                def _():
                    issue(kk + 1, 1 - b)

                @pl.when(kk < nch)
                def _():
                    process(b)

        pltpu.make_async_copy(ROWS[0], sh_acc.at[DST[0]], SEMS[0]).wait()
        pltpu.make_async_copy(ROWS[1], sh_acc.at[DST[1]], SEMS[1]).wait()

        plsc.subcore_barrier()

        @pl.when(sid < 10)
        def _():
            pltpu.sync_copy(sh_acc.at[pl.ds(r0, 1000)],
                            acc_hbm.at[cid, pl.ds(r0, 1000)])

    return body


def _sc_layer(F, COLS, CH, stage_hs, scale_heads, ei, hs, ad, z):
    mesh = plsc.VectorSubcoreMesh(core_axis_name="c", subcore_axis_name="s",
                                  num_cores=NC, num_subcores=NS)
    fn = pl.kernel(
        _make_sc_body(F, COLS, CH, stage_hs, scale_heads),
        out_type=jax.ShapeDtypeStruct((NC, N, COLS), _f32),
        mesh=mesh,
        scratch_types=(
            [pltpu.VMEM((CH,), _i32)] * 4
            + [pltpu.VMEM((CH, COLS), _f32)] * 2
            + [pltpu.VMEM((CH, 2 * NHEADS), _f32)] * 2
            + [pltpu.VMEM_SHARED((N, COLS), _f32)]
            + [pltpu.VMEM_SHARED((N, 2 * NHEADS), _f32)]
            + [pltpu.VMEM_SHARED((N, COLS if stage_hs else 1), _f32)]
            + [pltpu.SemaphoreType.DMA] * 8
        ),
        compiler_params=pltpu.CompilerParams(use_tc_tiling_on_sc=False),
    )
    return fn(ei, hs, ad, z)


def kernel(x, edge_index, W1, a1, W2, a2):
    ei = edge_index.astype(_i32)

    w1cat = jnp.transpose(W1, (1, 0, 2)).reshape(NFEAT, NHEADS * NHID)
    eye8 = jnp.eye(NHEADS, dtype=_f32)
    a_src = (a1[:, :NHID][..., None] * eye8[:, None, :]).reshape(NFEAT, NHEADS)
    a_dst = (a1[:, NHID:][..., None] * eye8[:, None, :]).reshape(NFEAT, NHEADS)
    rmat = jnp.kron(eye8, jnp.ones((1, NHID), dtype=_f32))
    ones16 = jnp.ones((1, NCLASS), dtype=_f32)
    a2s = a2[:NCLASS][:, None] * ones16
    a2d = a2[NCLASS:][:, None] * ones16

    z1 = jnp.zeros((N, W1COLS), _f32)
    z2 = jnp.zeros((N, W2COLS), _f32)

    hs1, ad1 = _tc1(x, w1cat, a_src, a_dst)
    acc1 = _sc_layer(NFEAT, W1COLS, CH1, False, True, ei, hs1, ad1, z1)
    hs2, ad2 = _tc2(acc1, W2, a2s, a2d, rmat)
    acc2 = _sc_layer(NCLASS, W2COLS, CH2, True, False, ei, hs2, ad2, z2)
    return _tc3(acc2)


# speedup vs baseline: 121.7829x; 1.2205x over previous
import jax
import jax.numpy as jnp
from jax import lax
from jax.experimental import pallas as pl
from jax.experimental.pallas import tpu as pltpu
from jax.experimental.pallas import tpu_sc as plsc

N = 10000
E = 320000
NFEAT = 128
NHID = 16
NCLASS = 16
NHEADS = 8
ALPHA = 0.2

NC, NS, L = 2, 16, 16
NW = NC * NS
CH1 = 80
CH2 = 128
W1COLS = NFEAT + NHID
W2COLS = 2 * NCLASS

_f32 = jnp.float32
_i32 = jnp.int32


def _leaky_exp(e):
    return jnp.exp(jnp.where(e >= 0, e, ALPHA * e))


def _tc1_body(x_ref, w_ref, as_ref, ad_ref, hs_ref, d_ref):
    h = jnp.dot(x_ref[...], w_ref[...], preferred_element_type=_f32)
    s = jnp.dot(h, as_ref[...], preferred_element_type=_f32)
    d = jnp.dot(h, ad_ref[...], preferred_element_type=_f32)
    hs_ref[...] = jnp.concatenate([h, s, s], axis=1)
    d_ref[...] = jnp.concatenate([d, d], axis=1)


def _tc1(x, w1cat, a_src, a_dst):
    B = 2000
    grid = (N // B,)
    return pl.pallas_call(
        _tc1_body,
        grid=grid,
        in_specs=[
            pl.BlockSpec((B, NFEAT), lambda i: (i, 0)),
            pl.BlockSpec((NFEAT, NFEAT), lambda i: (0, 0)),
            pl.BlockSpec((NFEAT, NHEADS), lambda i: (0, 0)),
            pl.BlockSpec((NFEAT, NHEADS), lambda i: (0, 0)),
        ],
        out_specs=[
            pl.BlockSpec((B, W1COLS), lambda i: (i, 0)),
            pl.BlockSpec((B, 2 * NHEADS), lambda i: (i, 0)),
        ],
        out_shape=[
            jax.ShapeDtypeStruct((N, W1COLS), _f32),
            jax.ShapeDtypeStruct((N, 2 * NHEADS), _f32),
        ],
    )(x, w1cat, a_src, a_dst)


def _tc2_body(acc_ref, w2_ref, a2s_ref, a2d_ref, r_ref, hs_ref, d_ref):
    o = acc_ref[0] + acc_ref[1]
    r = 1.0 / (o[:, NFEAT:NFEAT + NHEADS] + 1e-9)
    db = jnp.dot(r, r_ref[...], preferred_element_type=_f32)
    on = o[:, :NFEAT] * db
    el = jnp.where(on > 0, on, jnp.exp(jnp.minimum(on, 0.0)) - 1.0)
    h2 = jnp.dot(el, w2_ref[...], preferred_element_type=_f32)
    s2 = jnp.dot(h2, a2s_ref[...], preferred_element_type=_f32)
    d_ref[...] = jnp.dot(h2, a2d_ref[...], preferred_element_type=_f32)
    hs_ref[...] = jnp.concatenate([h2, s2], axis=1)


def _tc2(acc1, w2, a2s, a2d, rmat):
    B = 2000
    grid = (N // B,)
    return pl.pallas_call(
        _tc2_body,
        grid=grid,
        in_specs=[
            pl.BlockSpec((NC, B, W1COLS), lambda i: (0, i, 0)),
            pl.BlockSpec((NFEAT, NCLASS), lambda i: (0, 0)),
            pl.BlockSpec((NCLASS, NCLASS), lambda i: (0, 0)),
            pl.BlockSpec((NCLASS, NCLASS), lambda i: (0, 0)),
            pl.BlockSpec((NHEADS, NFEAT), lambda i: (0, 0)),
        ],
        out_specs=[
            pl.BlockSpec((B, W2COLS), lambda i: (i, 0)),
            pl.BlockSpec((B, NCLASS), lambda i: (i, 0)),
        ],
        out_shape=[
            jax.ShapeDtypeStruct((N, W2COLS), _f32),
            jax.ShapeDtypeStruct((N, NCLASS), _f32),
        ],
    )(acc1, w2, a2s, a2d, rmat)


def _tc3_body(acc_ref, out_ref):
    o = acc_ref[0] + acc_ref[1]
    out_ref[...] = o[:, :NCLASS] / (o[:, NCLASS:NCLASS + 1] + 1e-9)


def _tc3(acc2):
    B = 2000
    grid = (N // B,)
    return pl.pallas_call(
        _tc3_body,
        grid=grid,
        in_specs=[pl.BlockSpec((NC, B, W2COLS), lambda i: (0, i, 0))],
        out_specs=pl.BlockSpec((B, NCLASS), lambda i: (i, 0)),
        out_shape=jax.ShapeDtypeStruct((N, NCLASS), _f32),
    )(acc2)


def _make_sc_body(F, COLS, CH, stage_hs, scale_heads):
    def body(ei_hbm, hs_hbm, ad_hbm, z_hbm,
             acc_hbm,
             src0, src1, src2_, src3, dst0, dst1, dst2, dst3,
             rw0, rw1, gd0, gd1,
             sh_acc, sh_ad, sh_hs, semr0, semr1, sema0, sema1,
             sems0, sems1, semi0, semi1, semi2, semi3):
        cid = lax.axis_index("c")
        sid = lax.axis_index("s")
        wid = sid * NC + cid
        r0 = sid * 1000
        SRC = (src0, src1, src2_, src3)
        DST = (dst0, dst1, dst2, dst3)
        ROWS, GD = (rw0, rw1), (gd0, gd1)
        SEMR, SEMA = (semr0, semr1), (sema0, sema1)
        SEMS = (sems0, sems1)
        SEMI = (semi0, semi1, semi2, semi3)
        hs_src = sh_hs if stage_hs else hs_hbm
        totch = E // CH
        nb = totch // NW
        extra = totch - nb * NW
        nch = nb + jnp.where(wid < extra, 1, 0)
        nloop = nb + (-nb) % 4 + (4 if extra else 0)

        @pl.when(sid < 10)
        def _():
            pltpu.sync_copy(z_hbm.at[pl.ds(r0, 1000)],
                            sh_acc.at[pl.ds(r0, 1000)])
            pltpu.sync_copy(ad_hbm.at[pl.ds(r0, 1000)],
                            sh_ad.at[pl.ds(r0, 1000)])
            if stage_hs:
                pltpu.sync_copy(hs_hbm.at[pl.ds(r0, 1000)],
                                sh_hs.at[pl.ds(r0, 1000)])

        plsc.subcore_barrier()

        def idxissue(q, s):
            base = (wid + NW * q) * CH
            pltpu.async_copy(ei_hbm.at[0, pl.ds(base, CH)], SRC[s], SEMI[s])
            pltpu.async_copy(ei_hbm.at[1, pl.ds(base, CH)], DST[s], SEMI[s])

        def gissue(q, s, b):
            @pl.when(q >= 2)
            def _():
                pltpu.make_async_copy(ROWS[b], sh_acc.at[DST[(s + 2) % 4]],
                                      SEMS[b]).wait()

            base = (wid + NW * q) * CH
            pltpu.make_async_copy(ei_hbm.at[0, pl.ds(base, CH)], SRC[s],
                                  SEMI[s]).wait()
            pltpu.make_async_copy(ei_hbm.at[1, pl.ds(base, CH)], DST[s],
                                  SEMI[s]).wait()
            pltpu.async_copy(hs_src.at[SRC[s]], ROWS[b], SEMR[b])
            pltpu.async_copy(sh_ad.at[DST[s]], GD[b], SEMA[b])

        def process(s, b):
            pltpu.make_async_copy(sh_ad.at[DST[s]], GD[b], SEMA[b]).wait()
            pltpu.make_async_copy(hs_src.at[SRC[s]], ROWS[b], SEMR[b]).wait()

            @plsc.parallel_loop(0, CH, unroll=4)
            def _(i):
                row = ROWS[b].at[i]
                e = row[pl.ds(F, L)] + GD[b].at[i][...]
                w = _leaky_exp(e)
                row[pl.ds(F, L)] = w
                if scale_heads:
                    for j in range(NHEADS):
                        row[pl.ds(j * NHID, NHID)] = (
                            row[pl.ds(j * NHID, NHID)] * w[j])
                else:
                    row[pl.ds(0, L)] = row[pl.ds(0, L)] * w

            pltpu.async_copy(ROWS[b], sh_acc.at[DST[s]], SEMS[b], add=True)

        idxissue(0, 0)
        idxissue(1, 1)
        idxissue(2, 2)
        gissue(0, 0, 0)

        @pl.loop(0, nloop, step=4)
        def _(k):
            for b4 in range(4):
                kk = k + b4

                @pl.when(kk + 1 < nch)
                def _():
                    gissue(kk + 1, (b4 + 1) % 4, (b4 + 1) % 2)

                @pl.when(kk + 3 < nch)
                def _():
                    idxissue(kk + 3, (b4 + 3) % 4)

                @pl.when(kk < nch)
                def _():
                    process(b4 % 4, b4 % 2)

        pltpu.make_async_copy(ROWS[0], sh_acc.at[DST[0]], SEMS[0]).wait()
        pltpu.make_async_copy(ROWS[1], sh_acc.at[DST[1]], SEMS[1]).wait()

        plsc.subcore_barrier()

        @pl.when(sid < 10)
        def _():
            pltpu.sync_copy(sh_acc.at[pl.ds(r0, 1000)],
                            acc_hbm.at[cid, pl.ds(r0, 1000)])

    return body


def _sc_layer(F, COLS, CH, stage_hs, scale_heads, ei, hs, ad, z):
    mesh = plsc.VectorSubcoreMesh(core_axis_name="c", subcore_axis_name="s",
                                  num_cores=NC, num_subcores=NS)
    fn = pl.kernel(
        _make_sc_body(F, COLS, CH, stage_hs, scale_heads),
        out_type=jax.ShapeDtypeStruct((NC, N, COLS), _f32),
        mesh=mesh,
        scratch_types=(
            [pltpu.VMEM((CH,), _i32)] * 8
            + [pltpu.VMEM((CH, COLS), _f32)] * 2
            + [pltpu.VMEM((CH, 2 * NHEADS), _f32)] * 2
            + [pltpu.VMEM_SHARED((N, COLS), _f32)]
            + [pltpu.VMEM_SHARED((N, 2 * NHEADS), _f32)]
            + [pltpu.VMEM_SHARED((N, COLS if stage_hs else 1), _f32)]
            + [pltpu.SemaphoreType.DMA] * 10
        ),
        compiler_params=pltpu.CompilerParams(use_tc_tiling_on_sc=False),
    )
    return fn(ei, hs, ad, z)


def kernel(x, edge_index, W1, a1, W2, a2):
    ei = edge_index.astype(_i32)

    w1cat = jnp.transpose(W1, (1, 0, 2)).reshape(NFEAT, NHEADS * NHID)
    eye8 = jnp.eye(NHEADS, dtype=_f32)
    a_src = (a1[:, :NHID][..., None] * eye8[:, None, :]).reshape(NFEAT, NHEADS)
    a_dst = (a1[:, NHID:][..., None] * eye8[:, None, :]).reshape(NFEAT, NHEADS)
    rmat = jnp.kron(eye8, jnp.ones((1, NHID), dtype=_f32))
    ones16 = jnp.ones((1, NCLASS), dtype=_f32)
    a2s = a2[:NCLASS][:, None] * ones16
    a2d = a2[NCLASS:][:, None] * ones16

    z1 = jnp.zeros((N, W1COLS), _f32)
    z2 = jnp.zeros((N, W2COLS), _f32)

    hs1, ad1 = _tc1(x, w1cat, a_src, a_dst)
    acc1 = _sc_layer(NFEAT, W1COLS, CH1, False, True, ei, hs1, ad1, z1)
    hs2, ad2 = _tc2(acc1, W2, a2s, a2d, rmat)
    acc2 = _sc_layer(NCLASS, W2COLS, CH2, True, False, ei, hs2, ad2, z2)
    return _tc3(acc2)
